# baseline (speedup 1.0000x reference)
import functools

import jax
import jax.numpy as jnp
from jax import lax
from jax.experimental import pallas as pl
from jax.experimental.pallas import tpu as pltpu, tpu_sc as plsc

_E_TOTAL = 2 * 160000
_NWORK = 25
_PER_W = 160000 // _NWORK
_VECS = _PER_W // 16


def _pool_body(x_ref, o0, o1, o2, o3, o4):
    def pool(t):
        n = t.shape[0]
        m = t.reshape(n // 2, 512)
        return (m[:, :256] + m[:, 256:]) * 0.5

    v = x_ref[...]
    o0[...] = v
    p1 = pool(v)
    p2 = pool(p1)
    p3 = pool(p2)
    p4 = pool(p3)
    o1[...] = p1
    o2[...] = p2
    o3[...] = p3
    o4[...] = p4


def _make_edge_sc():
    mesh = plsc.VectorSubcoreMesh(core_axis_name="c", subcore_axis_name="s")

    @functools.partial(
        pl.kernel,
        mesh=mesh,
        out_type=[jax.ShapeDtypeStruct((2, 160000), jnp.int32)] * 5,
        scratch_types=[pltpu.VMEM((2, _PER_W), jnp.int32) for _ in range(5)],
    )
    def edge_sc(e_hbm, o0_hbm, o1_hbm, o2_hbm, o3_hbm, o4_hbm,
                buf, b1, b2, b3, b4):
        wid = lax.axis_index("s") * 2 + lax.axis_index("c")

        @pl.when(wid < _NWORK)
        def _():
            sl = pl.ds(wid * _PER_W, _PER_W)
            pltpu.sync_copy(e_hbm.at[:, sl], buf)

            def row_body(r):
                def body(i, carry):
                    s = pl.ds(i * 16, 16)
                    v = buf[r, s]
                    b1[r, s] = v >> 1
                    b2[r, s] = v >> 2
                    b3[r, s] = v >> 3
                    b4[r, s] = v >> 4
                    return carry
                lax.fori_loop(0, _VECS, body, 0, unroll=8)

            row_body(0)
            row_body(1)
            pltpu.sync_copy(buf, o0_hbm.at[:, sl])
            pltpu.sync_copy(b1, o1_hbm.at[:, sl])
            pltpu.sync_copy(b2, o2_hbm.at[:, sl])
            pltpu.sync_copy(b3, o3_hbm.at[:, sl])
            pltpu.sync_copy(b4, o4_hbm.at[:, sl])

    return edge_sc


def kernel(x, edge_index):
    e0, e1, e2, e3, e4 = _make_edge_sc()(edge_index)

    x0, x1, x2, x3, x4 = pl.pallas_call(
        _pool_body,
        out_shape=[
            jax.ShapeDtypeStruct((10000, 256), jnp.float32),
            jax.ShapeDtypeStruct((5000, 256), jnp.float32),
            jax.ShapeDtypeStruct((2500, 256), jnp.float32),
            jax.ShapeDtypeStruct((1250, 256), jnp.float32),
            jax.ShapeDtypeStruct((625, 256), jnp.float32),
        ],
    )(x)

    return (x0, x1, x2, x3, x4, e0, e1, e2, e3, e4)


# speedup vs baseline: 1.9125x; 1.9125x over previous
import jax
import jax.numpy as jnp
from jax.experimental import pallas as pl
from jax.experimental.pallas import tpu as pltpu

_G = 5
_XC = 10000 // _G
_EC = 160000 // _G


def _pool(t):
    n = t.shape[0]
    m = t.reshape(n // 2, 512)
    return (m[:, :256] + m[:, 256:]) * 0.5


def _body(x_ref, e_ref, o0, o1, o2, o3, o4, g0, g1, g2, g3, g4, x1buf):
    i = pl.program_id(0)
    v = x_ref[...]
    o0[...] = v
    p1 = _pool(v)
    o1[...] = p1
    off = pl.multiple_of(i * (_XC // 2), 8)
    x1buf[pl.ds(off, _XC // 2), :] = p1

    e = e_ref[...]
    g0[...] = e
    g1[...] = e >> 1
    g2[...] = e >> 2
    g3[...] = e >> 3
    g4[...] = e >> 4

    @pl.when(i == _G - 1)
    def _():
        t2 = _pool(x1buf[...])
        o2[...] = t2
        t3 = _pool(t2)
        o3[...] = t3
        o4[...] = _pool(t3)


def kernel(x, edge_index):
    outs = pl.pallas_call(
        _body,
        grid=(_G,),
        in_specs=[
            pl.BlockSpec((_XC, 256), lambda i: (i, 0)),
            pl.BlockSpec((2, _EC), lambda i: (0, i)),
        ],
        out_specs=[
            pl.BlockSpec((_XC, 256), lambda i: (i, 0)),
            pl.BlockSpec((_XC // 2, 256), lambda i: (i, 0)),
            pl.BlockSpec((2500, 256), lambda i: (0, 0)),
            pl.BlockSpec((1250, 256), lambda i: (0, 0)),
            pl.BlockSpec((625, 256), lambda i: (0, 0)),
            pl.BlockSpec((2, _EC), lambda i: (0, i)),
            pl.BlockSpec((2, _EC), lambda i: (0, i)),
            pl.BlockSpec((2, _EC), lambda i: (0, i)),
            pl.BlockSpec((2, _EC), lambda i: (0, i)),
            pl.BlockSpec((2, _EC), lambda i: (0, i)),
        ],
        out_shape=[
            jax.ShapeDtypeStruct((10000, 256), jnp.float32),
            jax.ShapeDtypeStruct((5000, 256), jnp.float32),
            jax.ShapeDtypeStruct((2500, 256), jnp.float32),
            jax.ShapeDtypeStruct((1250, 256), jnp.float32),
            jax.ShapeDtypeStruct((625, 256), jnp.float32),
            jax.ShapeDtypeStruct((2, 160000), jnp.int32),
            jax.ShapeDtypeStruct((2, 160000), jnp.int32),
            jax.ShapeDtypeStruct((2, 160000), jnp.int32),
            jax.ShapeDtypeStruct((2, 160000), jnp.int32),
            jax.ShapeDtypeStruct((2, 160000), jnp.int32),
        ],
        scratch_shapes=[pltpu.VMEM((5000, 256), jnp.float32)],
    )(x, edge_index)
    x0, x1, x2, x3, x4, e0, e1, e2, e3, e4 = outs
    return (x0, x1, x2, x3, x4, e0, e1, e2, e3, e4)


# speedup vs baseline: 1.9194x; 1.0036x over previous
import jax
import jax.numpy as jnp
from jax.experimental import pallas as pl


def _body(x_ref, e_ref, o0, o1, o2, o3, o4, g0, g1, g2, g3, g4):
    e = e_ref[...]
    g0[...] = e
    g1[...] = e >> 1
    g2[...] = e >> 2
    g3[...] = e >> 3
    g4[...] = e >> 4

    def pool(t):
        n = t.shape[0]
        m = t.reshape(n // 2, 512)
        return (m[:, :256] + m[:, 256:]) * 0.5

    v = x_ref[...]
    o0[...] = v
    p1 = pool(v)
    p2 = pool(p1)
    p3 = pool(p2)
    p4 = pool(p3)
    o1[...] = p1
    o2[...] = p2
    o3[...] = p3
    o4[...] = p4


def kernel(x, edge_index):
    outs = pl.pallas_call(
        _body,
        out_shape=[
            jax.ShapeDtypeStruct((10000, 256), jnp.float32),
            jax.ShapeDtypeStruct((5000, 256), jnp.float32),
            jax.ShapeDtypeStruct((2500, 256), jnp.float32),
            jax.ShapeDtypeStruct((1250, 256), jnp.float32),
            jax.ShapeDtypeStruct((625, 256), jnp.float32),
            jax.ShapeDtypeStruct((2, 160000), jnp.int32),
            jax.ShapeDtypeStruct((2, 160000), jnp.int32),
            jax.ShapeDtypeStruct((2, 160000), jnp.int32),
            jax.ShapeDtypeStruct((2, 160000), jnp.int32),
            jax.ShapeDtypeStruct((2, 160000), jnp.int32),
        ],
    )(x, edge_index)
    x0, x1, x2, x3, x4, e0, e1, e2, e3, e4 = outs
    return (x0, x1, x2, x3, x4, e0, e1, e2, e3, e4)
